# baseline (speedup 1.0000x reference)
import functools

import jax
import jax.numpy as jnp
from jax import lax
from jax.experimental import pallas as pl
from jax.experimental.pallas import tpu as pltpu
from jax.experimental.pallas import tpu_sc as plsc

N = 10000
D = 128
E = 320000

NC = 2
NS = 16
NW = NC * NS
CHUNK = 128
CPW = 79
EPW = CPW * CHUNK
E_PAD = NW * EPW
P = 5
PREAL = 2000
PROWS = 2048
ACC_ROWS = P * PROWS
RPS = PROWS // NS
PAD_DST = N + 8
DEG_W = 16


def _sc_segment_sum(x, src, dst):
    mesh = plsc.VectorSubcoreMesh(core_axis_name="c", subcore_axis_name="s")

    scratch = ([pltpu.VMEM_SHARED((PROWS, D), jnp.float32)] * P
               + [pltpu.VMEM_SHARED((PROWS, DEG_W), jnp.float32)] * P
               + [pltpu.VMEM((CHUNK,), jnp.int32),
                  pltpu.VMEM((CHUNK,), jnp.int32),
                  pltpu.VMEM((CHUNK,), jnp.int32),
                  pltpu.VMEM((CHUNK, D), jnp.float32),
                  pltpu.VMEM((CHUNK, DEG_W), jnp.float32),
                  pltpu.SemaphoreType.DMA])

    @functools.partial(
        pl.kernel,
        out_type=[
            jax.ShapeDtypeStruct((NC * ACC_ROWS, D), jnp.float32),
            jax.ShapeDtypeStruct((NC * ACC_ROWS, DEG_W), jnp.float32),
        ],
        mesh=mesh,
        scratch_types=scratch,
    )
    def sc_kernel(x_hbm, src_hbm, dst_hbm, agg_hbm, deg_hbm, *refs):
        aggs = refs[0:P]
        degs = refs[P:2 * P]
        sidx, didx, didx_p, rows, ones, sem = refs[2 * P:]
        cid = lax.axis_index("c")
        sid = lax.axis_index("s")
        wid = sid * NC + cid

        @pl.loop(0, CHUNK)
        def _(i):
            ones[i, pl.ds(0, 16)] = jnp.zeros((16,), jnp.float32)

            @pl.loop(0, D // 16)
            def _(j):
                rows[i, pl.ds(j * 16, 16)] = jnp.zeros((16,), jnp.float32)

        for p in range(P):
            pltpu.sync_copy(rows, aggs[p].at[pl.ds(sid * RPS, RPS)])
            pltpu.sync_copy(ones, degs[p].at[pl.ds(sid * RPS, RPS)])

        plsc.subcore_barrier()

        @pl.loop(0, CHUNK)
        def _(i):
            ones[i, pl.ds(0, 16)] = jnp.ones((16,), jnp.float32)

        base_w = wid * EPW

        @pl.loop(0, CPW)
        def _(c):
            b = base_w + c * CHUNK
            pltpu.sync_copy(src_hbm.at[pl.ds(b, CHUNK)], sidx)
            pltpu.sync_copy(dst_hbm.at[pl.ds(b, CHUNK)], didx)
            pltpu.async_copy(x_hbm.at[sidx], rows, sem).wait()
            for p in range(P):
                for j in range(CHUNK // 16):
                    d = didx[pl.ds(j * 16, 16)]
                    lo = d - p * PREAL
                    ok = (lo >= 0) & (lo < PREAL)
                    dump = PREAL + (d & 31)
                    didx_p[pl.ds(j * 16, 16)] = jnp.where(ok, lo, dump)
                pltpu.sync_copy(rows, aggs[p].at[didx_p], add=True)
                pltpu.sync_copy(ones, degs[p].at[didx_p], add=True)

        plsc.subcore_barrier()

        for p in range(P):
            r = sid * RPS
            o = cid * ACC_ROWS + p * PROWS + r
            pltpu.sync_copy(aggs[p].at[pl.ds(r, RPS)], rows)
            pltpu.sync_copy(rows, agg_hbm.at[pl.ds(o, RPS)])
            pltpu.sync_copy(degs[p].at[pl.ds(r, RPS)], ones)
            pltpu.sync_copy(ones, deg_hbm.at[pl.ds(o, RPS)])

    return sc_kernel(x, src, dst)


def _tc_combine(x, a0, a1, d0, d1, W_l, b_l, W_r):
    BLK = 1000
    dn = (((1,), (1,)), ((), ()))

    def body(x_ref, a0_ref, a1_ref, d0_ref, d1_ref, wl_ref, bl_ref, wr_ref,
             o_ref):
        agg = a0_ref[...] + a1_ref[...]
        deg = d0_ref[...][:, 0:1] + d1_ref[...][:, 0:1]
        agg = agg / jnp.maximum(deg, 1.0)
        h = lax.dot_general(agg, wl_ref[...], dn,
                            preferred_element_type=jnp.float32,
                            precision=lax.Precision.HIGHEST)
        h = h + lax.dot_general(x_ref[...], wr_ref[...], dn,
                                preferred_element_type=jnp.float32,
                                precision=lax.Precision.HIGHEST)
        o_ref[...] = jnp.maximum(h + bl_ref[...], 0.0)

    row_spec = pl.BlockSpec((BLK, D), lambda i: (i, 0))
    deg_spec = pl.BlockSpec((BLK, DEG_W), lambda i: (i, 0))
    full_spec = pl.BlockSpec((D, D), lambda i: (0, 0))
    bias_spec = pl.BlockSpec((1, D), lambda i: (0, 0))
    return pl.pallas_call(
        body,
        grid=(N // BLK,),
        in_specs=[row_spec, row_spec, row_spec, deg_spec, deg_spec,
                  full_spec, bias_spec, full_spec],
        out_specs=row_spec,
        out_shape=jax.ShapeDtypeStruct((N, D), jnp.float32),
    )(x, a0, a1, d0, d1, W_l, b_l, W_r)


def kernel(x, edge_index, W_l, b_l, W_r):
    src = edge_index[0].astype(jnp.int32)
    dst = edge_index[1].astype(jnp.int32)
    pad = E_PAD - E
    src = jnp.concatenate([src, jnp.zeros((pad,), jnp.int32)])
    dst = jnp.concatenate([dst, jnp.full((pad,), PAD_DST, jnp.int32)])
    agg_p, deg_p = _sc_segment_sum(x, src, dst)
    a = agg_p.reshape(NC, P, PROWS, D)[:, :, :PREAL].reshape(NC, N, D)
    d = deg_p.reshape(NC, P, PROWS, DEG_W)[:, :, :PREAL].reshape(NC, N, DEG_W)
    return _tc_combine(x, a[0], a[1], d[0], d[1], W_l, b_l.reshape(1, D), W_r)


# speedup vs baseline: 1.0552x; 1.0552x over previous
import functools

import jax
import jax.numpy as jnp
from jax import lax
from jax.experimental import pallas as pl
from jax.experimental.pallas import tpu as pltpu
from jax.experimental.pallas import tpu_sc as plsc

N = 10000
D = 128
E = 320000

NC = 2
NS = 16
NW = NC * NS
L = 16
CHUNK = 128
CPW = 79
EPW = CPW * CHUNK
E_PAD = NW * EPW
P = 5
PREAL = 2000
PROWS = 2048
ACC_ROWS = P * PROWS
RPS = PROWS // NS
PAD_DST = N + 8
LCAP = 256
DEGN = 10256


def _sc_segment_sum(x, src, dst):
    mesh = plsc.VectorSubcoreMesh(core_axis_name="c", subcore_axis_name="s")

    scratch = [
        pltpu.VMEM_SHARED((PROWS, D), jnp.float32),
        pltpu.VMEM_SHARED((PROWS, D), jnp.float32),
        pltpu.VMEM_SHARED((PROWS, D), jnp.float32),
        pltpu.VMEM_SHARED((PROWS, D), jnp.float32),
        pltpu.VMEM_SHARED((PROWS, D), jnp.float32),
        pltpu.VMEM((P * LCAP,), jnp.int32),
        pltpu.VMEM((P * LCAP,), jnp.int32),
        pltpu.VMEM((CHUNK,), jnp.int32),
        pltpu.VMEM((CHUNK,), jnp.int32),
        pltpu.VMEM((CHUNK,), jnp.int32),
        pltpu.VMEM((CHUNK,), jnp.int32),
        pltpu.VMEM((CHUNK, D), jnp.float32),
        pltpu.VMEM((8, L), jnp.int32),
        pltpu.VMEM((DEGN,), jnp.float32),
        pltpu.SemaphoreType.DMA,
    ]

    @functools.partial(
        pl.kernel,
        out_type=[jax.ShapeDtypeStruct((NC * ACC_ROWS, D), jnp.float32),
                  jax.ShapeDtypeStruct((NW, DEGN), jnp.float32)],
        mesh=mesh,
        scratch_types=scratch,
        compiler_params=pltpu.CompilerParams(needs_layout_passes=False),
    )
    def sc_kernel(x_hbm, src_hbm, dst_hbm, agg_hbm, degw_hbm,
                  acc0, acc1, acc2, acc3, acc4,
                  slist, dlist, schunk, dchunk, fsrc, fdst, rows, nfill,
                  degloc, sem):
        aggs = (acc0, acc1, acc2, acc3, acc4)
        cid = lax.axis_index("c")
        sid = lax.axis_index("s")
        wid = sid * NC + cid

        lane = jax.lax.iota(jnp.int32, L)

        @pl.loop(0, CHUNK)
        def _(i):
            @pl.loop(0, D // L)
            def _(j):
                rows[i, pl.ds(j * L, L)] = jnp.zeros((L,), jnp.float32)

        for p in range(P):
            pltpu.sync_copy(rows, aggs[p].at[pl.ds(sid * RPS, RPS)])
            nfill[p, pl.ds(0, L)] = jnp.zeros((L,), jnp.int32)

        @pl.loop(0, DEGN // L)
        def _(i):
            degloc[pl.ds(i * L, L)] = jnp.zeros((L,), jnp.float32)

        plsc.subcore_barrier()

        def flush(p, n_valid):
            for k in range(CHUNK // L):
                gl = lane + (k * L)
                keep = gl < n_valid
                sv = slist[pl.ds(p * LCAP + k * L, L)]
                dv = dlist[pl.ds(p * LCAP + k * L, L)]
                dump = PREAL + (gl & 31)
                fsrc[pl.ds(k * L, L)] = jnp.where(keep, sv, 0)
                fdst[pl.ds(k * L, L)] = jnp.where(keep, dv, dump)
            pltpu.async_copy(x_hbm.at[fsrc], rows, sem).wait()
            pltpu.sync_copy(rows, aggs[p].at[fdst], add=True)

        base_w = wid * EPW

        @pl.loop(0, CPW)
        def _(c):
            b = base_w + c * CHUNK
            pltpu.sync_copy(src_hbm.at[pl.ds(b, CHUNK)], schunk)
            pltpu.sync_copy(dst_hbm.at[pl.ds(b, CHUNK)], dchunk)
            for k in range(CHUNK // L):
                dd = dchunk[pl.ds(k * L, L)]
                plsc.addupdate_scatter(degloc, [dd], jnp.ones((L,), jnp.float32))
            for p in range(P):
                ptr = nfill[p, pl.ds(0, L)]
                for k in range(CHUNK // L):
                    d = dchunk[pl.ds(k * L, L)]
                    s = schunk[pl.ds(k * L, L)]
                    lo = d - p * PREAL
                    m = (lo >= 0) & (lo < PREAL)
                    mi = m.astype(jnp.int32)
                    pos = (p * LCAP) + ptr + jnp.cumsum(mi) - 1
                    plsc.store_scatter(slist, [pos], s, mask=m)
                    plsc.store_scatter(dlist, [pos], lo, mask=m)
                    ptr = ptr + plsc.all_reduce_population_count(m)
                nfill[p, pl.ds(0, L)] = ptr

                @pl.when(jnp.max(ptr) >= CHUNK)
                def _():
                    flush(p, jnp.full((L,), CHUNK, jnp.int32))
                    for k in range(CHUNK // L):
                        sv = slist[pl.ds(p * LCAP + CHUNK + k * L, L)]
                        dv = dlist[pl.ds(p * LCAP + CHUNK + k * L, L)]
                        slist[pl.ds(p * LCAP + k * L, L)] = sv
                        dlist[pl.ds(p * LCAP + k * L, L)] = dv
                    nfill[p, pl.ds(0, L)] = ptr - CHUNK

        for p in range(P):
            flush(p, nfill[p, pl.ds(0, L)])

        plsc.subcore_barrier()

        for p in range(P):
            r = sid * RPS
            o = cid * ACC_ROWS + p * PROWS + r
            pltpu.sync_copy(aggs[p].at[pl.ds(r, RPS)], rows)
            pltpu.sync_copy(rows, agg_hbm.at[pl.ds(o, RPS)])
        pltpu.sync_copy(degloc, degw_hbm.at[wid])

    return sc_kernel(x, src, dst)


def _tc_combine(x, a0, a1, g, W_l, b_l, W_r):
    BLK = 1000
    dn = (((1,), (1,)), ((), ()))

    def body(x_ref, a0_ref, a1_ref, g_ref, wl_ref, bl_ref, wr_ref,
             o_ref):
        agg = a0_ref[...] + a1_ref[...]
        deg = jnp.sum(g_ref[...], axis=1, keepdims=True)
        agg = agg / jnp.maximum(deg, 1.0)
        h = lax.dot_general(agg, wl_ref[...], dn,
                            preferred_element_type=jnp.float32,
                            precision=lax.Precision.HIGHEST)
        h = h + lax.dot_general(x_ref[...], wr_ref[...], dn,
                                preferred_element_type=jnp.float32,
                                precision=lax.Precision.HIGHEST)
        o_ref[...] = jnp.maximum(h + bl_ref[...], 0.0)

    row_spec = pl.BlockSpec((BLK, D), lambda i: (i, 0))
    deg_spec = pl.BlockSpec((BLK, NW), lambda i: (i, 0))
    full_spec = pl.BlockSpec((D, D), lambda i: (0, 0))
    bias_spec = pl.BlockSpec((1, D), lambda i: (0, 0))
    return pl.pallas_call(
        body,
        grid=(N // BLK,),
        in_specs=[row_spec, row_spec, row_spec, deg_spec,
                  full_spec, bias_spec, full_spec],
        out_specs=row_spec,
        out_shape=jax.ShapeDtypeStruct((N, D), jnp.float32),
    )(x, a0, a1, g, W_l, b_l, W_r)


def kernel(x, edge_index, W_l, b_l, W_r):
    src = edge_index[0].astype(jnp.int32)
    dst = edge_index[1].astype(jnp.int32)
    pad = E_PAD - E
    src = jnp.concatenate([src, jnp.zeros((pad,), jnp.int32)])
    dst = jnp.concatenate([dst, jnp.full((pad,), PAD_DST, jnp.int32)])
    agg_p, degw = _sc_segment_sum(x, src, dst)
    a = agg_p.reshape(NC, P, PROWS, D)[:, :, :PREAL].reshape(NC, N, D)
    g = degw[:, :N].T
    return _tc_combine(x, a[0], a[1], g, W_l, b_l.reshape(1, D), W_r)


# speedup vs baseline: 1.1653x; 1.1044x over previous
import functools

import jax
import jax.numpy as jnp
from jax import lax
from jax.experimental import pallas as pl
from jax.experimental.pallas import tpu as pltpu
from jax.experimental.pallas import tpu_sc as plsc

N = 10000
D = 128
E = 320000

NC = 2
NS = 16
NW = NC * NS
L = 16
CHUNK = 128
CPW = 79
EPW = CPW * CHUNK
E_PAD = NW * EPW
P = 5
PREAL = 2000
PROWS = 2048
ACC_ROWS = P * PROWS
RPS = PROWS // NS
PAD_DST = N + 8
LCAP = 256
DEGN = 10256


def _sc_segment_sum(x, src, dst):
    mesh = plsc.VectorSubcoreMesh(core_axis_name="c", subcore_axis_name="s")

    scratch = [
        pltpu.VMEM_SHARED((PROWS, D), jnp.float32),
        pltpu.VMEM_SHARED((PROWS, D), jnp.float32),
        pltpu.VMEM_SHARED((PROWS, D), jnp.float32),
        pltpu.VMEM_SHARED((PROWS, D), jnp.float32),
        pltpu.VMEM_SHARED((PROWS, D), jnp.float32),
        pltpu.VMEM((P * LCAP,), jnp.int32),
        pltpu.VMEM((P * LCAP,), jnp.int32),
        pltpu.VMEM((CHUNK,), jnp.int32),
        pltpu.VMEM((CHUNK,), jnp.int32),
        pltpu.VMEM((CHUNK,), jnp.int32),
        pltpu.VMEM((CHUNK,), jnp.int32),
        pltpu.VMEM((CHUNK,), jnp.int32),
        pltpu.VMEM((CHUNK,), jnp.int32),
        pltpu.VMEM((CHUNK, D), jnp.float32),
        pltpu.VMEM((8, L), jnp.int32),
        pltpu.VMEM((DEGN,), jnp.float32),
        pltpu.SemaphoreType.DMA,
        pltpu.SemaphoreType.DMA,
        pltpu.SemaphoreType.DMA,
    ]

    @functools.partial(
        pl.kernel,
        out_type=[jax.ShapeDtypeStruct((NC * ACC_ROWS, D), jnp.float32),
                  jax.ShapeDtypeStruct((NW, DEGN), jnp.float32)],
        mesh=mesh,
        scratch_types=scratch,
        compiler_params=pltpu.CompilerParams(needs_layout_passes=False),
    )
    def sc_kernel(x_hbm, src_hbm, dst_hbm, agg_hbm, degw_hbm,
                  acc0, acc1, acc2, acc3, acc4,
                  slist, dlist, schunk0, dchunk0, schunk1, dchunk1,
                  fsrc, fdst, rows, nfill, degloc, sem, isem0, isem1):
        aggs = (acc0, acc1, acc2, acc3, acc4)
        cid = lax.axis_index("c")
        sid = lax.axis_index("s")
        wid = sid * NC + cid

        lane = jax.lax.iota(jnp.int32, L)

        @pl.loop(0, CHUNK)
        def _(i):
            @pl.loop(0, D // L)
            def _(j):
                rows[i, pl.ds(j * L, L)] = jnp.zeros((L,), jnp.float32)

        for p in range(P):
            pltpu.sync_copy(rows, aggs[p].at[pl.ds(sid * RPS, RPS)])
            nfill[p, pl.ds(0, L)] = jnp.zeros((L,), jnp.int32)

        @pl.loop(0, DEGN // L)
        def _(i):
            degloc[pl.ds(i * L, L)] = jnp.zeros((L,), jnp.float32)

        plsc.subcore_barrier()

        def flush(p, n_valid):
            for k in range(CHUNK // L):
                gl = lane + (k * L)
                keep = gl < n_valid
                sv = slist[pl.ds(p * LCAP + k * L, L)]
                dv = dlist[pl.ds(p * LCAP + k * L, L)]
                dump = PREAL + (gl & 31)
                fsrc[pl.ds(k * L, L)] = jnp.where(keep, sv, 0)
                fdst[pl.ds(k * L, L)] = jnp.where(keep, dv, dump)
            pltpu.async_copy(x_hbm.at[fsrc], rows, sem).wait()
            pltpu.sync_copy(rows, aggs[p].at[fdst], add=True)

        base_w = wid * EPW

        def route(schunk, dchunk):
            for k in range(CHUNK // L):
                dd = dchunk[pl.ds(k * L, L)]
                plsc.addupdate_scatter(degloc, [dd],
                                       jnp.ones((L,), jnp.float32))
            for p in range(P):
                ptr = nfill[p, pl.ds(0, L)]
                for k in range(CHUNK // L):
                    d = dchunk[pl.ds(k * L, L)]
                    s = schunk[pl.ds(k * L, L)]
                    lo = d - p * PREAL
                    m = (lo >= 0) & (lo < PREAL)
                    mi = m.astype(jnp.int32)
                    pos = (p * LCAP) + ptr + jnp.cumsum(mi) - 1
                    plsc.store_scatter(slist, [pos], s, mask=m)
                    plsc.store_scatter(dlist, [pos], lo, mask=m)
                    ptr = ptr + plsc.all_reduce_population_count(m)
                nfill[p, pl.ds(0, L)] = ptr

                @pl.when(jnp.max(ptr) >= CHUNK)
                def _():
                    flush(p, jnp.full((L,), CHUNK, jnp.int32))
                    for k in range(CHUNK // L):
                        sv = slist[pl.ds(p * LCAP + CHUNK + k * L, L)]
                        dv = dlist[pl.ds(p * LCAP + CHUNK + k * L, L)]
                        slist[pl.ds(p * LCAP + k * L, L)] = sv
                        dlist[pl.ds(p * LCAP + k * L, L)] = dv
                    nfill[p, pl.ds(0, L)] = ptr - CHUNK

        def issue(c, sbuf, dbuf, isem):
            b = base_w + c * CHUNK
            pltpu.async_copy(src_hbm.at[pl.ds(b, CHUNK)], sbuf, isem)
            pltpu.async_copy(dst_hbm.at[pl.ds(b, CHUNK)], dbuf, isem)

        def wait(sbuf, dbuf, isem):
            pltpu.make_async_copy(src_hbm.at[pl.ds(0, CHUNK)], sbuf,
                                  isem).wait()
            pltpu.make_async_copy(dst_hbm.at[pl.ds(0, CHUNK)], dbuf,
                                  isem).wait()

        issue(0, schunk0, dchunk0, isem0)

        @pl.loop(0, CPW // 2)
        def _(h):
            c = h * 2
            wait(schunk0, dchunk0, isem0)
            issue(c + 1, schunk1, dchunk1, isem1)
            route(schunk0, dchunk0)
            wait(schunk1, dchunk1, isem1)

            @pl.when(c + 2 < CPW - 1)
            def _():
                issue(c + 2, schunk0, dchunk0, isem0)
            route(schunk1, dchunk1)

        issue(CPW - 1, schunk0, dchunk0, isem0)
        wait(schunk0, dchunk0, isem0)
        route(schunk0, dchunk0)

        for p in range(P):
            flush(p, nfill[p, pl.ds(0, L)])

        plsc.subcore_barrier()

        for p in range(P):
            r = sid * RPS
            o = cid * ACC_ROWS + p * PROWS + r
            pltpu.sync_copy(aggs[p].at[pl.ds(r, RPS)], rows)
            pltpu.sync_copy(rows, agg_hbm.at[pl.ds(o, RPS)])
        pltpu.sync_copy(degloc, degw_hbm.at[wid])

    return sc_kernel(x, src, dst)


def _tc_combine(x, a0, a1, g, W_l, b_l, W_r):
    BLK = 1000
    dn = (((1,), (1,)), ((), ()))

    def body(x_ref, a0_ref, a1_ref, g_ref, wl_ref, bl_ref, wr_ref,
             o_ref):
        agg = a0_ref[...] + a1_ref[...]
        deg = jnp.sum(g_ref[...], axis=1, keepdims=True)
        agg = agg / jnp.maximum(deg, 1.0)
        h = lax.dot_general(agg, wl_ref[...], dn,
                            preferred_element_type=jnp.float32,
                            precision=lax.Precision.HIGHEST)
        h = h + lax.dot_general(x_ref[...], wr_ref[...], dn,
                                preferred_element_type=jnp.float32,
                                precision=lax.Precision.HIGHEST)
        o_ref[...] = jnp.maximum(h + bl_ref[...], 0.0)

    row_spec = pl.BlockSpec((BLK, D), lambda i: (i, 0))
    deg_spec = pl.BlockSpec((BLK, NW), lambda i: (i, 0))
    full_spec = pl.BlockSpec((D, D), lambda i: (0, 0))
    bias_spec = pl.BlockSpec((1, D), lambda i: (0, 0))
    return pl.pallas_call(
        body,
        grid=(N // BLK,),
        in_specs=[row_spec, row_spec, row_spec, deg_spec,
                  full_spec, bias_spec, full_spec],
        out_specs=row_spec,
        out_shape=jax.ShapeDtypeStruct((N, D), jnp.float32),
    )(x, a0, a1, g, W_l, b_l, W_r)


def kernel(x, edge_index, W_l, b_l, W_r):
    src = edge_index[0].astype(jnp.int32)
    dst = edge_index[1].astype(jnp.int32)
    pad = E_PAD - E
    src = jnp.concatenate([src, jnp.zeros((pad,), jnp.int32)])
    dst = jnp.concatenate([dst, jnp.full((pad,), PAD_DST, jnp.int32)])
    agg_p, degw = _sc_segment_sum(x, src, dst)
    a = agg_p.reshape(NC, P, PROWS, D)[:, :, :PREAL].reshape(NC, N, D)
    g = degw[:, :N].T
    return _tc_combine(x, a[0], a[1], g, W_l, b_l.reshape(1, D), W_r)
